# baseline (speedup 1.0000x reference)
import functools

import jax
import jax.numpy as jnp
from jax import lax
from jax.experimental import pallas as pl
from jax.experimental.pallas import tpu as pltpu
from jax.experimental.pallas import tpu_sc as plsc

VOCAB = 100000
EMBED_DIM = 64
BATCH = 4096
HIST = 50

NUM_CORES = 2
NUM_SUBCORES = 16
NW = NUM_CORES * NUM_SUBCORES
B_PER_W = BATCH // NW
NCHUNK = B_PER_W

_mesh = plsc.VectorSubcoreMesh(core_axis_name="c", subcore_axis_name="s")


@functools.partial(
    pl.kernel,
    mesh=_mesh,
    out_type=jax.ShapeDtypeStruct((BATCH, HIST, EMBED_DIM), jnp.float32),
    compiler_params=pltpu.CompilerParams(use_tc_tiling_on_sc=False),
    scratch_types=[
        pltpu.VMEM((B_PER_W, HIST), jnp.int32),
        pltpu.VMEM((2, HIST, EMBED_DIM), jnp.float32),
        pltpu.SemaphoreType.DMA,
        pltpu.SemaphoreType.DMA,
        pltpu.SemaphoreType.DMA,
        pltpu.SemaphoreType.DMA,
    ],
)
def _emb_lookup(x_hbm, table_hbm, out_hbm, idx_v, rows_v, gsem0, gsem1,
                wsem0, wsem1):
    wid = lax.axis_index("s") * NUM_CORES + lax.axis_index("c")
    bbase = wid * B_PER_W

    pltpu.sync_copy(x_hbm.at[pl.ds(bbase, B_PER_W)], idx_v)

    gsems = (gsem0, gsem1)
    wsems = (wsem0, wsem1)

    def gather(j, b):
        pltpu.async_copy(
            table_hbm.at[idx_v.at[j]], rows_v.at[b], gsems[b]
        )

    gather(0, 0)
    gather(1, 1)

    def chunk_body(j, _):
        for b in range(2):
            @pl.when(j % 2 == b)
            def _():
                pltpu.make_async_copy(
                    table_hbm.at[idx_v.at[0]], rows_v.at[b],
                    gsems[b]
                ).wait()
                pltpu.async_copy(
                    rows_v.at[b],
                    out_hbm.at[bbase + j],
                    wsems[b],
                )

        @pl.when(j + 2 < NCHUNK)
        def _():
            for b in range(2):
                @pl.when(j % 2 == b)
                def _():
                    pltpu.make_async_copy(
                        rows_v.at[b],
                        out_hbm.at[bbase],
                        wsems[b],
                    ).wait()
                    gather(j + 2, b)
        return 0

    lax.fori_loop(0, NCHUNK, chunk_body, 0)

    for b in range(2):
        pltpu.make_async_copy(
            rows_v.at[b], out_hbm.at[bbase], wsems[b]
        ).wait()


def kernel(x, table):
    return _emb_lookup(x.astype(jnp.int32), table)
